# baseline (speedup 1.0000x reference)
import functools

import jax
import jax.numpy as jnp
from jax import lax
from jax.experimental import pallas as pl
from jax.experimental.pallas import tpu as pltpu
from jax.experimental.pallas import tpu_sc as plsc

HIDDEN = 4096
NUM_EXPERTS = 64
TOP_K = 8
NEG_INF = float("-inf")
NS = 4

T_TOTAL = 4 * 4096
N_WORKERS = 32
TPW = T_TOTAL // N_WORKERS
CH = 128
NCH = TPW // CH
NG = CH // 16


def _matmul_body(*refs):
    x_refs = refs[:NS]
    wt_ref, b_ref, logits_ref = refs[NS:]
    sub = x_refs[0].shape[0]
    for j in range(NS):
        logits = (
            jnp.dot(x_refs[j][...], wt_ref[...], preferred_element_type=jnp.float32)
            + b_ref[...]
        )
        logits_ref[pl.ds(j * sub, sub), :] = logits


def _sc_topk_body(logits_hbm, w_hbm, i_hbm, tile_v, wout_v, iout_v):
    cid = lax.axis_index("c")
    sid = lax.axis_index("s")
    wid = sid * 2 + cid
    lane = lax.iota(jnp.int32, 16)
    neg_inf = jnp.full((16,), NEG_INF, jnp.float32)

    def chunk_body(c, carry):
        base_tok = wid * TPW + c * CH
        pltpu.sync_copy(logits_hbm.at[pl.ds(base_tok, CH), :], tile_v)

        def group_body(g, carry2):
            rows = g * 16 + lane
            work = [
                plsc.load_gather(tile_v, [rows, jnp.full((16,), e, jnp.int32)])
                for e in range(NUM_EXPERTS)
            ]
            vals = []
            idxs = []
            for _ in range(TOP_K):
                m = work[0]
                for e in range(1, NUM_EXPERTS):
                    m = jnp.maximum(m, work[e])
                idx = jnp.full((16,), NUM_EXPERTS, jnp.int32)
                for e in range(NUM_EXPERTS - 1, -1, -1):
                    idx = jnp.where(work[e] == m, e, idx)
                vals.append(m)
                idxs.append(idx)
                for e in range(NUM_EXPERTS):
                    work[e] = jnp.where(idx == e, neg_inf, work[e])
            exps = [jnp.exp(v - vals[0]) for v in vals]
            tot = exps[0]
            for k in range(1, TOP_K):
                tot = tot + exps[k]
            out_rows = g * 16 + lane
            for k in range(TOP_K):
                col = jnp.full((16,), k, jnp.int32)
                plsc.store_scatter(wout_v, [out_rows, col], exps[k] / tot)
                plsc.store_scatter(iout_v, [out_rows, col], idxs[k])
            return carry2

        lax.fori_loop(0, NG, group_body, 0, unroll=False)
        pltpu.sync_copy(wout_v, w_hbm.at[pl.ds(base_tok, CH), :])
        pltpu.sync_copy(iout_v, i_hbm.at[pl.ds(base_tok, CH), :])
        return carry

    lax.fori_loop(0, NCH, chunk_body, 0, unroll=False)


@jax.jit
def _router(hidden_states, W, b):
    B, S, H = hidden_states.shape
    T = B * S
    x = hidden_states.reshape(T, H)
    wt = W.T
    b2 = b.reshape(1, NUM_EXPERTS)
    block_tokens = 1024
    sub = block_tokens // NS

    grid = (T // block_tokens,)
    xspecs = [
        pl.BlockSpec(
            (sub, H), functools.partial(lambda j, t: (NS * t + j, 0), j)
        )
        for j in range(NS)
    ]
    logits = pl.pallas_call(
        _matmul_body,
        grid=grid,
        in_specs=xspecs
        + [
            pl.BlockSpec((H, NUM_EXPERTS), lambda t: (0, 0)),
            pl.BlockSpec((1, NUM_EXPERTS), lambda t: (0, 0)),
        ],
        out_specs=pl.BlockSpec((block_tokens, NUM_EXPERTS), lambda t: (t, 0)),
        out_shape=jax.ShapeDtypeStruct((T, NUM_EXPERTS), jnp.float32),
        compiler_params=pltpu.CompilerParams(
            dimension_semantics=("parallel",),
        ),
    )(*([x] * NS), wt, b2)

    sc_topk = pl.kernel(
        _sc_topk_body,
        out_type=[
            jax.ShapeDtypeStruct((T, TOP_K), jnp.float32),
            jax.ShapeDtypeStruct((T, TOP_K), jnp.int32),
        ],
        mesh=plsc.VectorSubcoreMesh(core_axis_name="c", subcore_axis_name="s"),
        compiler_params=pltpu.CompilerParams(needs_layout_passes=False),
        scratch_types=[
            pltpu.VMEM((CH, NUM_EXPERTS), jnp.float32),
            pltpu.VMEM((CH, TOP_K), jnp.float32),
            pltpu.VMEM((CH, TOP_K), jnp.int32),
        ],
    )
    weights, indices = sc_topk(logits)

    return (
        weights.reshape(B, S, TOP_K),
        indices.reshape(B, S, TOP_K),
        logits.reshape(B, S, NUM_EXPERTS),
    )


def kernel(hidden_states, W, b):
    return _router(hidden_states, W, b)


# speedup vs baseline: 1.4901x; 1.4901x over previous
import functools

import jax
import jax.numpy as jnp
from jax.experimental import pallas as pl
from jax.experimental.pallas import tpu as pltpu

HIDDEN = 4096
NUM_EXPERTS = 64
TOP_K = 8
NEG_INF = float("-inf")
NS = 4


def _topk_softmax(logits):
    work = logits.T
    eid = jax.lax.broadcasted_iota(jnp.int32, work.shape, 0).astype(jnp.float32)
    vals = []
    idxs = []
    for _ in range(TOP_K):
        m = jnp.max(work, axis=0, keepdims=True)
        idx = jnp.min(
            jnp.where(work == m, eid, float(NUM_EXPERTS)), axis=0, keepdims=True
        )
        vals.append(m)
        idxs.append(idx)
        work = jnp.where(eid == idx, NEG_INF, work)
    v = jnp.concatenate(vals, axis=0)
    i = jnp.concatenate(idxs, axis=0)
    e = jnp.exp(v - v[0:1, :])
    w = e / jnp.sum(e, axis=0, keepdims=True)
    return w.T, i.T


def _router_body(*refs):
    x_refs = refs[:NS]
    wt_ref, b_ref, logits_ref, w_ref, i_ref = refs[NS:]
    sub = x_refs[0].shape[0]
    for j in range(NS):
        logits = (
            jnp.dot(x_refs[j][...], wt_ref[...], preferred_element_type=jnp.float32)
            + b_ref[...]
        )
        rows = pl.ds(j * sub, sub)
        logits_ref[rows, :] = logits
        w, i = _topk_softmax(logits)
        w_ref[rows, :] = w
        i_ref[rows, :] = i.astype(jnp.int32)


@functools.partial(jax.jit, static_argnames=("block_tokens",))
def _router(hidden_states, W, b, block_tokens=1024):
    B, S, H = hidden_states.shape
    T = B * S
    x = hidden_states.reshape(T, H)
    wt = W.T
    b2 = b.reshape(1, NUM_EXPERTS)
    sub = block_tokens // NS

    grid = (T // block_tokens,)
    xspecs = [
        pl.BlockSpec(
            (sub, H), functools.partial(lambda j, t: (NS * t + j, 0), j)
        )
        for j in range(NS)
    ]
    logits, weights, indices = pl.pallas_call(
        _router_body,
        grid=grid,
        in_specs=xspecs
        + [
            pl.BlockSpec((H, NUM_EXPERTS), lambda t: (0, 0)),
            pl.BlockSpec((1, NUM_EXPERTS), lambda t: (0, 0)),
        ],
        out_specs=[
            pl.BlockSpec((block_tokens, NUM_EXPERTS), lambda t: (t, 0)),
            pl.BlockSpec((block_tokens, TOP_K), lambda t: (t, 0)),
            pl.BlockSpec((block_tokens, TOP_K), lambda t: (t, 0)),
        ],
        out_shape=[
            jax.ShapeDtypeStruct((T, NUM_EXPERTS), jnp.float32),
            jax.ShapeDtypeStruct((T, TOP_K), jnp.float32),
            jax.ShapeDtypeStruct((T, TOP_K), jnp.int32),
        ],
        compiler_params=pltpu.CompilerParams(
            dimension_semantics=("parallel",),
        ),
    )(*([x] * NS), wt, b2)

    return (
        weights.reshape(B, S, TOP_K),
        indices.reshape(B, S, TOP_K),
        logits.reshape(B, S, NUM_EXPERTS),
    )


def kernel(hidden_states, W, b):
    return _router(hidden_states, W, b)


# speedup vs baseline: 1.4935x; 1.0023x over previous
import functools

import jax
import jax.numpy as jnp
from jax.experimental import pallas as pl
from jax.experimental.pallas import tpu as pltpu

HIDDEN = 4096
NUM_EXPERTS = 64
TOP_K = 8
NEG_INF = float("-inf")
NS = 8


def _topk_softmax(logits):
    work = logits.T
    eid = jax.lax.broadcasted_iota(jnp.int32, work.shape, 0).astype(jnp.float32)
    vals = []
    idxs = []
    for _ in range(TOP_K):
        m = jnp.max(work, axis=0, keepdims=True)
        idx = jnp.min(
            jnp.where(work == m, eid, float(NUM_EXPERTS)), axis=0, keepdims=True
        )
        vals.append(m)
        idxs.append(idx)
        work = jnp.where(eid == idx, NEG_INF, work)
    v = jnp.concatenate(vals, axis=0)
    i = jnp.concatenate(idxs, axis=0)
    e = jnp.exp(v - v[0:1, :])
    w = e / jnp.sum(e, axis=0, keepdims=True)
    return w.T, i.T


def _router_body(*refs):
    x_refs = refs[:NS]
    wt_ref, b_ref, logits_ref, w_ref, i_ref = refs[NS:]
    sub = x_refs[0].shape[0]
    for j in range(NS):
        logits = (
            jnp.dot(x_refs[j][...], wt_ref[...], preferred_element_type=jnp.float32)
            + b_ref[...]
        )
        rows = pl.ds(j * sub, sub)
        logits_ref[rows, :] = logits
        w, i = _topk_softmax(logits)
        w_ref[rows, :] = w
        i_ref[rows, :] = i.astype(jnp.int32)


@functools.partial(jax.jit, static_argnames=("block_tokens",))
def _router(hidden_states, W, b, block_tokens=1024):
    B, S, H = hidden_states.shape
    T = B * S
    x = hidden_states.reshape(T, H)
    wt = W.T
    b2 = b.reshape(1, NUM_EXPERTS)
    sub = block_tokens // NS

    grid = (T // block_tokens,)
    xspecs = [
        pl.BlockSpec(
            (sub, H), functools.partial(lambda j, t: (NS * t + j, 0), j)
        )
        for j in range(NS)
    ]
    logits, weights, indices = pl.pallas_call(
        _router_body,
        grid=grid,
        in_specs=xspecs
        + [
            pl.BlockSpec((H, NUM_EXPERTS), lambda t: (0, 0)),
            pl.BlockSpec((1, NUM_EXPERTS), lambda t: (0, 0)),
        ],
        out_specs=[
            pl.BlockSpec((block_tokens, NUM_EXPERTS), lambda t: (t, 0)),
            pl.BlockSpec((block_tokens, TOP_K), lambda t: (t, 0)),
            pl.BlockSpec((block_tokens, TOP_K), lambda t: (t, 0)),
        ],
        out_shape=[
            jax.ShapeDtypeStruct((T, NUM_EXPERTS), jnp.float32),
            jax.ShapeDtypeStruct((T, TOP_K), jnp.float32),
            jax.ShapeDtypeStruct((T, TOP_K), jnp.int32),
        ],
        compiler_params=pltpu.CompilerParams(
            dimension_semantics=("parallel",),
        ),
    )(*([x] * NS), wt, b2)

    return (
        weights.reshape(B, S, TOP_K),
        indices.reshape(B, S, TOP_K),
        logits.reshape(B, S, NUM_EXPERTS),
    )


def kernel(hidden_states, W, b):
    return _router(hidden_states, W, b)
